# baseline (speedup 1.0000x reference)
import functools

import jax
import jax.numpy as jnp
from jax import lax
from jax.experimental import pallas as pl
from jax.experimental.pallas import tpu as pltpu
from jax.experimental.pallas import tpu_sc as plsc

MAX_SPAN = 255
QUERY_LENGTH = 254
KEY_LENGTH = 256
DEPTH = 512
TABLE_ROWS = MAX_SPAN * 2 - 1

_NUM_CORES = 2
_NUM_SUBCORES = 16
_NUM_WORKERS = _NUM_CORES * _NUM_SUBCORES
_Q_PER_WORKER = -(-QUERY_LENGTH // _NUM_WORKERS)

_ROT_ROWS = [504, 504, 504, 504, 504, 504, 496, 496]
_PACKED_ROWS = sum(_ROT_ROWS)


def _pack_body(w_ref, o_ref):
    base = 0
    for r, n in enumerate(_ROT_ROWS):
        o_ref[base : base + n] = w_ref[r : r + n]
        base += n


def _pack_rotated_tables(weight):
    return pl.pallas_call(
        _pack_body,
        in_specs=[pl.BlockSpec((TABLE_ROWS, DEPTH), lambda: (0, 0))],
        out_specs=pl.BlockSpec((_PACKED_ROWS, DEPTH), lambda: (0, 0)),
        out_shape=jax.ShapeDtypeStruct((_PACKED_ROWS, DEPTH), jnp.float32),
    )(weight)


def _make_sc_kernel():
    mesh = plsc.VectorSubcoreMesh(core_axis_name="c", subcore_axis_name="s")

    @functools.partial(
        pl.kernel,
        mesh=mesh,
        out_type=jax.ShapeDtypeStruct(
            (QUERY_LENGTH, KEY_LENGTH, DEPTH), jnp.float32
        ),
        scratch_types=[
            pltpu.VMEM_SHARED((_PACKED_ROWS, DEPTH), jnp.float32),
            pltpu.SemaphoreType.DMA,
        ],
    )
    def sc_kernel(w8_hbm, out_hbm, shared, sem):
        cid = lax.axis_index("c")
        sid = lax.axis_index("s")
        wid = sid * _NUM_CORES + cid

        stripe = 256
        tail = _PACKED_ROWS - (_NUM_SUBCORES - 1) * stripe

        @pl.when(sid < _NUM_SUBCORES - 1)
        def _load():
            pltpu.sync_copy(
                w8_hbm.at[pl.ds(sid * stripe, stripe)],
                shared.at[pl.ds(sid * stripe, stripe)],
            )

        @pl.when(sid == _NUM_SUBCORES - 1)
        def _load_tail():
            base = (_NUM_SUBCORES - 1) * stripe
            pltpu.sync_copy(
                w8_hbm.at[pl.ds(base, tail)],
                shared.at[pl.ds(base, tail)],
            )

        plsc.subcore_barrier()

        copies = []
        for t in range(_Q_PER_WORKER):
            q = wid + _NUM_WORKERS * t
            qc = jnp.minimum(q, QUERY_LENGTH - 1)
            s = (MAX_SPAN - 2) - qc
            r = lax.rem(s, 8)
            a = s - r
            off = r * _ROT_ROWS[0] - jnp.maximum(r - 6, 0) * 8
            src = pl.multiple_of(off + a, 8)
            desc = pltpu.make_async_copy(
                shared.at[pl.ds(src, KEY_LENGTH), :], out_hbm.at[qc], sem
            )
            copies.append((q, desc))

            @pl.when(q < QUERY_LENGTH)
            def _start(desc=desc):
                desc.start()

        for q, desc in copies:

            @pl.when(q < QUERY_LENGTH)
            def _wait(desc=desc):
                desc.wait()

    return sc_kernel


def kernel(weight):
    return _make_sc_kernel()(_pack_rotated_tables(weight))
